# baseline (speedup 1.0000x reference)
import functools

import jax
import jax.numpy as jnp
from jax import lax
from jax.experimental import pallas as pl
from jax.experimental.pallas import tpu as pltpu
from jax.experimental.pallas import tpu_sc as plsc

_NC, _NS = 2, 16
_NW = _NC * _NS
_CH = 128


@functools.lru_cache(maxsize=None)
def _make_lookup(V, DP, N):
    n_ch = N // _CH
    cpw = n_ch // _NW
    mesh = plsc.VectorSubcoreMesh(core_axis_name="c", subcore_axis_name="s")

    nb = 4

    @functools.partial(
        pl.kernel,
        mesh=mesh,
        out_type=jax.ShapeDtypeStruct((N, DP), jnp.float32),
        scratch_types=[
            pltpu.VMEM((cpw, _CH), jnp.int32),
        ] + [pltpu.VMEM((_CH, DP), jnp.float32) for _ in range(nb)]
          + [pltpu.SemaphoreType.DMA for _ in range(nb)],
    )
    def lookup_kernel(idx_hbm, table_hbm, out_hbm, idx_v, *bufsem):
        bufs, sems = bufsem[:nb], bufsem[nb:]
        wid = lax.axis_index("s") * _NC + lax.axis_index("c")
        c0 = wid * cpw
        pltpu.sync_copy(idx_hbm.at[pl.ds(c0, cpw)], idx_v)

        def gather(c, j):
            pltpu.async_copy(
                table_hbm.at[idx_v.at[c, pl.ds(0, _CH)]], bufs[j], sems[j])

        for j in range(nb - 1):
            gather(j, j)

        def body(r, carry):
            i0 = r * nb
            for j in range(nb):
                i = i0 + j
                pltpu.make_async_copy(
                    table_hbm.at[idx_v.at[i, pl.ds(0, _CH)]],
                    bufs[j], sems[j]).wait()

                @pl.when(i + nb - 1 < cpw)
                def _():
                    gather(i + nb - 1, (j + nb - 1) % nb)

                pltpu.sync_copy(
                    bufs[j], out_hbm.at[pl.ds((c0 + i) * _CH, _CH)])
            return carry

        lax.fori_loop(0, cpw // nb, body, 0)

    return lookup_kernel


def kernel(indices, table):
    B0, B1 = indices.shape
    V, D = table.shape
    DP = 128
    N = B0 * B1
    ns = 5
    Np = N // ns
    table_p = jnp.pad(table, ((0, 0), (0, DP - D)))
    idx_flat = indices.astype(jnp.int32).reshape(N // _CH, _CH)
    lk = _make_lookup(V, DP, Np)
    parts = [
        lk(idx_flat[i * (Np // _CH):(i + 1) * (Np // _CH)], table_p)[:, :D]
        for i in range(ns)
    ]
    return jnp.concatenate(parts, axis=0).reshape(B0, B1, D)


# speedup vs baseline: 1.3322x; 1.3322x over previous
import functools

import jax
import jax.numpy as jnp
from jax import lax
from jax.experimental import pallas as pl
from jax.experimental.pallas import tpu as pltpu
from jax.experimental.pallas import tpu_sc as plsc

_NC, _NS = 2, 16
_NW = _NC * _NS
_CH = 128


@functools.lru_cache(maxsize=None)
def _make_lookup(V, DP, N):
    n_ch = N // _CH
    cpw = n_ch // _NW
    mesh = plsc.VectorSubcoreMesh(core_axis_name="c", subcore_axis_name="s")

    nb = 3

    @functools.partial(
        pl.kernel,
        mesh=mesh,
        out_type=jax.ShapeDtypeStruct((N, DP), jnp.float32),
        scratch_types=[
            pltpu.VMEM((cpw, _CH), jnp.int32),
        ] + [pltpu.VMEM((_CH, DP), jnp.float32) for _ in range(nb)]
          + [pltpu.SemaphoreType.DMA for _ in range(nb)],
    )
    def lookup_kernel(idx_hbm, table_hbm, out_hbm, idx_v, *bufsem):
        bufs, sems = bufsem[:nb], bufsem[nb:]
        wid = lax.axis_index("s") * _NC + lax.axis_index("c")
        c0 = wid * cpw
        pltpu.sync_copy(idx_hbm.at[pl.ds(c0, cpw)], idx_v)

        def gather(c, j):
            pltpu.async_copy(
                table_hbm.at[idx_v.at[c, pl.ds(0, _CH)]], bufs[j], sems[j])

        for j in range(nb - 1):
            gather(j, j)

        def body(r, carry):
            i0 = r * nb
            for j in range(nb):
                i = i0 + j
                pltpu.make_async_copy(
                    table_hbm.at[idx_v.at[i, pl.ds(0, _CH)]],
                    bufs[j], sems[j]).wait()

                @pl.when(i + nb - 1 < cpw)
                def _():
                    gather(i + nb - 1, (j + nb - 1) % nb)

                pltpu.sync_copy(
                    bufs[j], out_hbm.at[pl.ds((c0 + i) * _CH, _CH)])
            return carry

        lax.fori_loop(0, cpw // nb, body, 0)

        for i in range((cpw // nb) * nb, cpw):
            j = i % nb
            pltpu.make_async_copy(
                table_hbm.at[idx_v.at[i, pl.ds(0, _CH)]],
                bufs[j], sems[j]).wait()
            pltpu.sync_copy(bufs[j], out_hbm.at[pl.ds((c0 + i) * _CH, _CH)])

    return lookup_kernel


def kernel(indices, table):
    B0, B1 = indices.shape
    V, D = table.shape
    DP = 128
    N = B0 * B1
    table_p = jnp.pad(table, ((0, 0), (0, DP - D)))
    idx_flat = indices.astype(jnp.int32).reshape(N // _CH, _CH)
    out = _make_lookup(V, DP, N)(idx_flat, table_p)
    return out[:, :D].reshape(B0, B1, D)


# speedup vs baseline: 1.3335x; 1.0010x over previous
import functools

import jax
import jax.numpy as jnp
from jax import lax
from jax.experimental import pallas as pl
from jax.experimental.pallas import tpu as pltpu
from jax.experimental.pallas import tpu_sc as plsc

_NC, _NS = 2, 16
_NW = _NC * _NS
_CH = 128


@functools.lru_cache(maxsize=None)
def _make_lookup(V, DP, N):
    n_ch = N // _CH
    cpw = n_ch // _NW
    mesh = plsc.VectorSubcoreMesh(core_axis_name="c", subcore_axis_name="s")

    @functools.partial(
        pl.kernel,
        mesh=mesh,
        out_type=jax.ShapeDtypeStruct((N, DP), jnp.float32),
        scratch_types=[
            pltpu.VMEM((cpw, _CH), jnp.int32),
            pltpu.VMEM((_CH, DP), jnp.float32),
            pltpu.VMEM((_CH, DP), jnp.float32),
            pltpu.SemaphoreType.DMA,
            pltpu.SemaphoreType.DMA,
        ],
    )
    def lookup_kernel(idx_hbm, table_hbm, out_hbm,
                      idx_v, buf_a, buf_b, sem_a, sem_b):
        wid = lax.axis_index("s") * _NC + lax.axis_index("c")
        c0 = wid * cpw
        pltpu.sync_copy(idx_hbm.at[pl.ds(c0, cpw)], idx_v)

        pltpu.async_copy(
            table_hbm.at[idx_v.at[0, pl.ds(0, _CH)]], buf_a, sem_a)

        def body(i, carry):
            ca = 2 * i
            pltpu.async_copy(
                table_hbm.at[idx_v.at[ca + 1, pl.ds(0, _CH)]], buf_b, sem_b)
            pltpu.make_async_copy(
                table_hbm.at[idx_v.at[ca, pl.ds(0, _CH)]], buf_a, sem_a
            ).wait()
            pltpu.sync_copy(buf_a, out_hbm.at[pl.ds((c0 + ca) * _CH, _CH)])

            @pl.when(i < cpw // 2 - 1)
            def _():
                pltpu.async_copy(
                    table_hbm.at[idx_v.at[ca + 2, pl.ds(0, _CH)]],
                    buf_a, sem_a)

            pltpu.make_async_copy(
                table_hbm.at[idx_v.at[ca + 1, pl.ds(0, _CH)]], buf_b, sem_b
            ).wait()
            pltpu.sync_copy(
                buf_b, out_hbm.at[pl.ds((c0 + ca + 1) * _CH, _CH)])
            return carry

        lax.fori_loop(0, cpw // 2, body, 0)

    return lookup_kernel


def kernel(indices, table):
    B0, B1 = indices.shape
    V, D = table.shape
    DP = 128
    N = B0 * B1
    table_p = jnp.pad(table, ((0, 0), (0, DP - D)))
    idx_flat = indices.astype(jnp.int32).reshape(N // _CH, _CH)
    out = _make_lookup(V, DP, N)(idx_flat, table_p)
    return out[:, :D].reshape(B0, B1, D)
